# baseline (speedup 1.0000x reference)
import functools

import jax
import jax.numpy as jnp
from jax import lax
from jax.experimental import pallas as pl
from jax.experimental.pallas import tpu as pltpu
from jax.experimental.pallas import tpu_sc as plsc

_B, _P, _O, _D = 16384, 97, 13, 128
_NW = 32
_BPW = _B // _NW
_CH = 128
_NST = _BPW // _CH
_L = 16

_mesh = plsc.VectorSubcoreMesh(core_axis_name="c", subcore_axis_name="s")


@functools.partial(
    pl.kernel,
    mesh=_mesh,
    out_type=jax.ShapeDtypeStruct((_B, 4, _D), jnp.float32),
    scratch_types=[
        pltpu.VMEM((_BPW,), jnp.int32),
        pltpu.VMEM((_BPW,), jnp.int32),
        pltpu.SMEM((_BPW,), jnp.int32),
        pltpu.VMEM_SHARED((_P, _D), jnp.float32),
        pltpu.VMEM_SHARED((_B,), jnp.int32),
        pltpu.VMEM((_O, _D), jnp.float32),
        pltpu.VMEM((_CH, _D), jnp.float32),
        pltpu.VMEM((_CH, _D), jnp.float32),
        pltpu.VMEM((_CH, _D), jnp.float32),
        pltpu.VMEM((_CH, _D), jnp.float32),
        pltpu.VMEM((_CH, _D), jnp.float32),
        pltpu.VMEM((_CH, _D), jnp.float32),
        pltpu.VMEM((_CH, _D), jnp.float32),
        pltpu.SemaphoreType.DMA,
        pltpu.SemaphoreType.DMA,
        pltpu.SemaphoreType.DMA,
        pltpu.SemaphoreType.DMA,
        pltpu.SemaphoreType.DMA,
    ],
)
def _former(x_hbm, op_hbm, y_hbm, num_emb_hbm, op_emb_hbm, out_hbm,
            xv, yv, ops, table_v, opidx_sp, tab13,
            bx0, bx1, bo0, bo1, by0, by1, beq,
            gs0, gs1, ws0, ws1, wse):
    sid = lax.axis_index("s")
    wid = sid * 2 + lax.axis_index("c")
    base = wid * _BPW
    pltpu.sync_copy(x_hbm.at[pl.ds(base, _BPW)], xv)
    pltpu.sync_copy(y_hbm.at[pl.ds(base, _BPW)], yv)
    pltpu.sync_copy(op_emb_hbm, tab13)

    @pl.when(sid == 0)
    def _stage_number_table():
        pltpu.sync_copy(num_emb_hbm, table_v)

    @pl.when(sid == 1)
    def _stage_op_indices():
        pltpu.sync_copy(op_hbm, opidx_sp)

    def _fill_eq(i, _):
        for c in range(_D // _L):
            beq[i, pl.ds(c * _L, _L)] = tab13[0, pl.ds(c * _L, _L)]
        return 0
    lax.fori_loop(0, _CH, _fill_eq, 0)

    plsc.subcore_barrier()
    pltpu.sync_copy(opidx_sp.at[pl.ds(base, _BPW)], ops)

    bxs, bos, bys = (bx0, bx1), (bo0, bo1), (by0, by1)
    gsems = (gs0, gs1)
    wsems = (ws0, ws1)
    gd = [None] * _NST
    wd = [None] * _NST

    def _gathers(t):
        p = t % 2
        sl = pl.ds(t * _CH, _CH)
        gd[t] = (
            pltpu.async_copy(table_v.at[xv.at[sl]], bxs[p], gsems[p]),
            pltpu.async_copy(table_v.at[yv.at[sl]], bys[p], gsems[p]),
        )

    def _assemble_op(t):
        bo = bos[t % 2]

        def body(i, _):
            r = ops[t * _CH + i]
            for c in range(_D // _L):
                bo[i, pl.ds(c * _L, _L)] = tab13[r, pl.ds(c * _L, _L)]
            return 0
        lax.fori_loop(0, _CH, body, 0)

    def _writes(t):
        p = t % 2
        rows = pl.ds(base + t * _CH, _CH)
        wd[t] = (
            pltpu.async_copy(bxs[p], out_hbm.at[rows, 0], wsems[p]),
            pltpu.async_copy(bos[p], out_hbm.at[rows, 1], wsems[p]),
            pltpu.async_copy(bys[p], out_hbm.at[rows, 2], wsems[p]),
            pltpu.async_copy(beq, out_hbm.at[rows, 3], wse),
        )

    _gathers(0)
    _assemble_op(0)
    for t in range(_NST):
        if t + 1 < _NST:
            if t >= 1:
                for d in wd[t - 1][:3]:
                    d.wait()
            _gathers(t + 1)
            _assemble_op(t + 1)
        for d in gd[t]:
            d.wait()
        _writes(t)
    for t in (_NST - 2, _NST - 1):
        for d in wd[t][:3]:
            d.wait()
    for t in range(_NST):
        wd[t][3].wait()


def kernel(x_idx, op_idx, y_idx, number_emb, op_emb):
    return _former(x_idx.astype(jnp.int32), op_idx.astype(jnp.int32),
                   y_idx.astype(jnp.int32), number_emb, op_emb)


# speedup vs baseline: 1.2748x; 1.2748x over previous
import functools

import jax
import jax.numpy as jnp
from jax import lax
from jax.experimental import pallas as pl
from jax.experimental.pallas import tpu as pltpu
from jax.experimental.pallas import tpu_sc as plsc

_B, _P, _O, _D = 16384, 97, 13, 128
_NW = 32
_BPW = _B // _NW
_CH = 128
_NST = _BPW // _CH
_L = 16

_mesh = plsc.VectorSubcoreMesh(core_axis_name="c", subcore_axis_name="s")


@functools.partial(
    pl.kernel,
    mesh=_mesh,
    out_type=jax.ShapeDtypeStruct((_B, 4, _D), jnp.float32),
    scratch_types=[
        pltpu.VMEM((_BPW,), jnp.int32),
        pltpu.VMEM((_BPW,), jnp.int32),
        pltpu.VMEM((_BPW,), jnp.int32),
        pltpu.VMEM((1, _D), jnp.float32),
        pltpu.VMEM_SHARED((_P + _O, _D), jnp.float32),
        pltpu.VMEM((_CH, _D), jnp.float32),
        pltpu.VMEM((_CH, _D), jnp.float32),
        pltpu.VMEM((_CH, _D), jnp.float32),
        pltpu.VMEM((_CH, _D), jnp.float32),
        pltpu.VMEM((_CH, _D), jnp.float32),
        pltpu.VMEM((_CH, _D), jnp.float32),
        pltpu.VMEM((_CH, _D), jnp.float32),
        pltpu.SemaphoreType.DMA,
        pltpu.SemaphoreType.DMA,
        pltpu.SemaphoreType.DMA,
        pltpu.SemaphoreType.DMA,
        pltpu.SemaphoreType.DMA,
    ],
)
def _former(x_hbm, op_hbm, y_hbm, num_emb_hbm, op_emb_hbm, out_hbm,
            xv, ov, yv, eqrow, table_v,
            bx0, bx1, bo0, bo1, by0, by1, beq,
            gs0, gs1, ws0, ws1, wse):
    sid = lax.axis_index("s")
    wid = sid * 2 + lax.axis_index("c")
    base = wid * _BPW
    pltpu.sync_copy(x_hbm.at[pl.ds(base, _BPW)], xv)
    pltpu.sync_copy(op_hbm.at[pl.ds(base, _BPW)], ov)
    pltpu.sync_copy(y_hbm.at[pl.ds(base, _BPW)], yv)
    @pl.when(sid == 0)
    def _stage_number_table():
        pltpu.sync_copy(num_emb_hbm, table_v.at[pl.ds(0, _P)])
    @pl.when(sid == 1)
    def _stage_op_table():
        pltpu.sync_copy(op_emb_hbm, table_v.at[pl.ds(_P, _O)])
    pltpu.sync_copy(op_emb_hbm.at[pl.ds(0, 1)], eqrow)
    for j in range(_BPW // _L):
        ov[pl.ds(j * _L, _L)] = ov[pl.ds(j * _L, _L)] + _P

    def _fill_eq(i, _):
        for c in range(_D // _L):
            beq[i, pl.ds(c * _L, _L)] = eqrow[0, pl.ds(c * _L, _L)]
        return 0
    lax.fori_loop(0, _CH, _fill_eq, 0)
    plsc.subcore_barrier()

    bxs, bos, bys = (bx0, bx1), (bo0, bo1), (by0, by1)
    gsems = (gs0, gs1)
    wsems = (ws0, ws1)
    gd = [None] * _NST
    wd = [None] * _NST

    def _gathers(t):
        p = t % 2
        sl = pl.ds(t * _CH, _CH)
        gd[t] = (
            pltpu.async_copy(table_v.at[xv.at[sl]], bxs[p], gsems[p]),
            pltpu.async_copy(table_v.at[ov.at[sl]], bos[p], gsems[p]),
            pltpu.async_copy(table_v.at[yv.at[sl]], bys[p], gsems[p]),
        )

    def _writes(t):
        p = t % 2
        rows = pl.ds(base + t * _CH, _CH)
        wd[t] = (
            pltpu.async_copy(bxs[p], out_hbm.at[rows, 0], wsems[p]),
            pltpu.async_copy(bos[p], out_hbm.at[rows, 1], wsems[p]),
            pltpu.async_copy(bys[p], out_hbm.at[rows, 2], wsems[p]),
            pltpu.async_copy(beq, out_hbm.at[rows, 3], wse),
        )

    _gathers(0)
    for t in range(_NST):
        if t + 1 < _NST:
            if t >= 1:
                for d in wd[t - 1][:3]:
                    d.wait()
            _gathers(t + 1)
        for d in gd[t]:
            d.wait()
        _writes(t)
    for t in (_NST - 2, _NST - 1):
        for d in wd[t][:3]:
            d.wait()
    for t in range(_NST):
        wd[t][3].wait()


def kernel(x_idx, op_idx, y_idx, number_emb, op_emb):
    return _former(x_idx.astype(jnp.int32), op_idx.astype(jnp.int32),
                   y_idx.astype(jnp.int32), number_emb, op_emb)


# speedup vs baseline: 1.3793x; 1.0820x over previous
import functools

import jax
import jax.numpy as jnp
from jax import lax
from jax.experimental import pallas as pl
from jax.experimental.pallas import tpu as pltpu
from jax.experimental.pallas import tpu_sc as plsc

_B, _P, _O, _D = 16384, 97, 13, 128
_NW = 32
_BPW = _B // _NW
_CH = 128
_NST = _BPW // _CH
_L = 16

_mesh = plsc.VectorSubcoreMesh(core_axis_name="c", subcore_axis_name="s")


@functools.partial(
    pl.kernel,
    mesh=_mesh,
    out_type=jax.ShapeDtypeStruct((_B, 4, _D), jnp.float32),
    scratch_types=[
        pltpu.VMEM((_BPW,), jnp.int32),
        pltpu.VMEM((_BPW,), jnp.int32),
        pltpu.VMEM((_BPW,), jnp.int32),
        pltpu.VMEM((_CH,), jnp.int32),
        pltpu.VMEM_SHARED((_P + _O, _D), jnp.float32),
        pltpu.VMEM((_CH, _D), jnp.float32),
        pltpu.VMEM((_CH, _D), jnp.float32),
        pltpu.VMEM((_CH, _D), jnp.float32),
        pltpu.VMEM((_CH, _D), jnp.float32),
        pltpu.VMEM((_CH, _D), jnp.float32),
        pltpu.VMEM((_CH, _D), jnp.float32),
        pltpu.VMEM((_CH, _D), jnp.float32),
        pltpu.SemaphoreType.DMA,
        pltpu.SemaphoreType.DMA,
        pltpu.SemaphoreType.DMA,
        pltpu.SemaphoreType.DMA,
        pltpu.SemaphoreType.DMA,
    ],
)
def _former(x_hbm, op_hbm, y_hbm, num_emb_hbm, op_emb_hbm, out_hbm,
            xv, ov, yv, eqi, table_v,
            bx0, bx1, bo0, bo1, by0, by1, beq,
            gs0, gs1, ws0, ws1, wse):
    sid = lax.axis_index("s")
    wid = sid * 2 + lax.axis_index("c")
    base = wid * _BPW
    pltpu.sync_copy(x_hbm.at[pl.ds(base, _BPW)], xv)
    pltpu.sync_copy(op_hbm.at[pl.ds(base, _BPW)], ov)
    pltpu.sync_copy(y_hbm.at[pl.ds(base, _BPW)], yv)
    @pl.when(sid == 0)
    def _stage_number_table():
        pltpu.sync_copy(num_emb_hbm, table_v.at[pl.ds(0, _P)])
    @pl.when(sid == 1)
    def _stage_op_table():
        pltpu.sync_copy(op_emb_hbm, table_v.at[pl.ds(_P, _O)])
    for j in range(_BPW // _L):
        ov[pl.ds(j * _L, _L)] = ov[pl.ds(j * _L, _L)] + _P
    for j in range(_CH // _L):
        eqi[pl.ds(j * _L, _L)] = jnp.full((_L,), _P, jnp.int32)
    plsc.subcore_barrier()

    pltpu.async_copy(table_v.at[eqi], beq, gs0).wait()

    bxs, bos, bys = (bx0, bx1), (bo0, bo1), (by0, by1)
    gsems = (gs0, gs1)
    wsems = (ws0, ws1)
    gd = [None] * _NST
    wd = [None] * _NST

    def _gathers(t):
        p = t % 2
        sl = pl.ds(t * _CH, _CH)
        gd[t] = (
            pltpu.async_copy(table_v.at[xv.at[sl]], bxs[p], gsems[p]),
            pltpu.async_copy(table_v.at[ov.at[sl]], bos[p], gsems[p]),
            pltpu.async_copy(table_v.at[yv.at[sl]], bys[p], gsems[p]),
        )

    def _writes(t):
        p = t % 2
        rows = pl.ds(base + t * _CH, _CH)
        wd[t] = (
            pltpu.async_copy(bxs[p], out_hbm.at[rows, 0], wsems[p]),
            pltpu.async_copy(bos[p], out_hbm.at[rows, 1], wsems[p]),
            pltpu.async_copy(bys[p], out_hbm.at[rows, 2], wsems[p]),
            pltpu.async_copy(beq, out_hbm.at[rows, 3], wse),
        )

    _gathers(0)
    for t in range(_NST):
        if t + 1 < _NST:
            if t >= 1:
                for d in wd[t - 1][:3]:
                    d.wait()
            _gathers(t + 1)
        for d in gd[t]:
            d.wait()
        _writes(t)
    for t in (_NST - 2, _NST - 1):
        for d in wd[t][:3]:
            d.wait()
    for t in range(_NST):
        wd[t][3].wait()


def kernel(x_idx, op_idx, y_idx, number_emb, op_emb):
    return _former(x_idx.astype(jnp.int32), op_idx.astype(jnp.int32),
                   y_idx.astype(jnp.int32), number_emb, op_emb)
